# baseline (speedup 1.0000x reference)
import functools

import jax
import jax.numpy as jnp
from jax import lax
from jax.experimental import pallas as pl
from jax.experimental.pallas import tpu as pltpu
from jax.experimental.pallas import tpu_sc as plsc

N = 10000
D = 128
NC = 2
NS = 16
NW = NC * NS
CHK = 128
CH = 84
EPW = CH * CHK
EPAD = NW * EPW
RPT = 632
NPAD = NS * RPT
DUMMY = N
BN = 1000
GRID = N // BN


def _kan_math(z, params_ref, wse_ref, wso_ref, wb_ref):
    g0 = params_ref[0]
    g1 = params_ref[1]
    a = params_ref[2]
    d0 = z - g0
    d1 = z - g1
    e0 = jnp.exp(a * d0 * d0)
    e1 = jnp.exp(a * d1 * d1)
    sil = z * jax.nn.sigmoid(z)
    o = jnp.dot(e0, wse_ref[...])
    o = o + jnp.dot(e1, wso_ref[...])
    o = o + jnp.dot(sil, wb_ref[...])
    return o


def _mean_norm(pp_ref, cc_ref):
    s = pp_ref[0] + pp_ref[1]
    c = jnp.maximum(cc_ref[0] + cc_ref[1], 1.0)
    h = s / c
    mu = jnp.mean(h, axis=1, keepdims=True)
    d = h - mu
    var = jnp.mean(d * d, axis=1, keepdims=True)
    return d * lax.rsqrt(var + 1e-5)


def _kan_body(params_ref, z_ref, wse_ref, wso_ref, wb_ref, out_ref):
    out_ref[...] = _kan_math(z_ref[...], params_ref, wse_ref, wso_ref, wb_ref)


def _norm_kan_body(params_ref, pp_ref, cc_ref, wse_ref, wso_ref, wb_ref, out_ref):
    z = _mean_norm(pp_ref, cc_ref)
    out_ref[...] = _kan_math(z, params_ref, wse_ref, wso_ref, wb_ref)


def _norm_kan_lsm_body(params_ref, pp_ref, cc_ref, wse_ref, wso_ref, wb_ref, out_ref):
    z = _mean_norm(pp_ref, cc_ref)
    o = _kan_math(z, params_ref, wse_ref, wso_ref, wb_ref)
    m = jnp.max(o, axis=1, keepdims=True)
    e = o - m
    out_ref[...] = e - jnp.log(jnp.sum(jnp.exp(e), axis=1, keepdims=True))


_W_SPEC = pl.BlockSpec((D, D), lambda i: (0, 0))
_ROW_SPEC = pl.BlockSpec((BN, D), lambda i: (i, 0))
_P_SPEC = pl.BlockSpec(memory_space=pltpu.SMEM)


def _kan_call(params, z, wse, wso, wb):
    return pl.pallas_call(
        _kan_body,
        grid=(GRID,),
        in_specs=[_P_SPEC, _ROW_SPEC, _W_SPEC, _W_SPEC, _W_SPEC],
        out_specs=_ROW_SPEC,
        out_shape=jax.ShapeDtypeStruct((N, D), jnp.float32),
    )(params, z, wse, wso, wb)


def _norm_kan_call(body, params, pp, cc, wse, wso, wb):
    return pl.pallas_call(
        body,
        grid=(GRID,),
        in_specs=[
            _P_SPEC,
            pl.BlockSpec((NC, BN, D), lambda i: (0, i, 0)),
            pl.BlockSpec((NC, BN, 1), lambda i: (0, i, 0)),
            _W_SPEC, _W_SPEC, _W_SPEC,
        ],
        out_specs=_ROW_SPEC,
        out_shape=jax.ShapeDtypeStruct((N, D), jnp.float32),
    )(params, pp, cc, wse, wso, wb)


@functools.lru_cache(maxsize=1)
def _make_seg():
    mesh = plsc.VectorSubcoreMesh(
        core_axis_name="c", subcore_axis_name="s", num_cores=NC, num_subcores=NS
    )

    @functools.partial(
        pl.kernel,
        out_type=(
            jax.ShapeDtypeStruct((NC, NPAD, D), jnp.float32),
            jax.ShapeDtypeStruct((NC * NPAD,), jnp.float32),
        ),
        mesh=mesh,
        scratch_types=[
            pltpu.VMEM((CH // 2, CHK), jnp.int32),
            pltpu.VMEM((CH // 2, CHK), jnp.int32),
            pltpu.VMEM((CHK, D), jnp.float32),
            pltpu.VMEM((CHK, D), jnp.float32),
            pltpu.VMEM((CHK,), jnp.float32),
            pltpu.VMEM((RPT,), jnp.float32),
            pltpu.VMEM_SHARED((NPAD, D), jnp.float32),
            pltpu.VMEM_SHARED((NPAD,), jnp.float32),
            pltpu.SemaphoreType.DMA,
            pltpu.SemaphoreType.DMA,
        ],
    )
    def seg(m_hbm, src_hbm, dst_hbm, za_hbm, zc_hbm, ones_hbm,
            agg_out, cnt_out,
            src_v, dst_v, rows_a, rows_b, ones_v, cstg_v, agg_sp, cnt_sp,
            sem_a, sem_b):
        cid = lax.axis_index("c")
        sid = lax.axis_index("s")
        w = cid * NS + sid
        r0 = sid * RPT
        pltpu.sync_copy(za_hbm.at[pl.ds(r0, RPT)], agg_sp.at[pl.ds(r0, RPT)])
        pltpu.sync_copy(zc_hbm.at[pl.ds(r0, RPT)], cstg_v)
        pltpu.sync_copy(cstg_v, cnt_sp.at[pl.ds(r0, RPT)])
        pltpu.sync_copy(ones_hbm, ones_v)
        plsc.subcore_barrier()

        HH = CH // 2
        for h in range(2):
            pltpu.sync_copy(src_hbm.at[w, h], src_v)
            pltpu.sync_copy(dst_hbm.at[w, h], dst_v)

            def body(j, carry):
                pltpu.async_copy(m_hbm.at[src_v.at[j]], rows_a, sem_a).wait()
                pltpu.sync_copy(rows_a, agg_sp.at[dst_v.at[j]], add=True)
                pltpu.sync_copy(ones_v, cnt_sp.at[dst_v.at[j]], add=True)
                return carry

            lax.fori_loop(0, HH, body, 0)
        plsc.subcore_barrier()
        pltpu.sync_copy(agg_sp.at[pl.ds(r0, RPT)], agg_out.at[cid, pl.ds(r0, RPT)])
        pltpu.sync_copy(cnt_sp.at[pl.ds(r0, RPT)], cstg_v)
        pltpu.sync_copy(cstg_v, cnt_out.at[pl.ds(cid * NPAD + r0, RPT)])

    return seg


def _params_of(grid):
    g0 = grid[0]
    g1 = grid[-1]
    denom = (g1 - g0) / (grid.shape[0] - 1)
    return jnp.stack([g0, g1, -1.0 / (denom * denom)])


def kernel(x, edge_index, grid1, Ws1, Wb1, grid2, Ws2, Wb2, grid_out, Ws_out, Wb_out):
    src, dst = edge_index[0], edge_index[1]
    loops = jnp.arange(N, dtype=jnp.int32)
    pad = EPAD - (src.shape[0] + N)
    srcs = jnp.concatenate(
        [src, loops, jnp.zeros((pad,), jnp.int32)]).reshape(NW, 2, CH // 2, CHK)
    dsts = jnp.concatenate(
        [dst, loops, jnp.full((pad,), DUMMY, jnp.int32)]).reshape(NW, 2, CH // 2, CHK)
    za = jnp.zeros((NPAD, D), jnp.float32)
    zc = jnp.zeros((NPAD,), jnp.float32)
    ones = jnp.ones((CHK,), jnp.float32)
    seg = _make_seg()

    m1 = _kan_call(_params_of(grid1), x, Ws1[0::2], Ws1[1::2], Wb1)
    agg1, cnt1 = seg(m1, srcs, dsts, za, zc, ones)
    m2 = _norm_kan_call(_norm_kan_body, _params_of(grid2), agg1,
                        cnt1.reshape(NC, NPAD, 1), Ws2[0::2], Ws2[1::2], Wb2)
    agg2, cnt2 = seg(m2, srcs, dsts, za, zc, ones)
    return _norm_kan_call(_norm_kan_lsm_body, _params_of(grid_out), agg2,
                          cnt2.reshape(NC, NPAD, 1), Ws_out[0::2], Ws_out[1::2], Wb_out)


# speedup vs baseline: 2.4875x; 2.4875x over previous
import functools

import jax
import jax.numpy as jnp
from jax import lax
from jax.experimental import pallas as pl
from jax.experimental.pallas import tpu as pltpu
from jax.experimental.pallas import tpu_sc as plsc

N = 10000
D = 128
NC = 2
NS = 16
NW = NC * NS
CH = 81
EPW = CH * 128
EPAD = NW * EPW
RPT = 632
NPAD = NS * RPT
DUMMY = N
BN = 1000
GRID = N // BN


def _kan_math(z, params_ref, wse_ref, wso_ref, wb_ref):
    g0 = params_ref[0]
    g1 = params_ref[1]
    a = params_ref[2]
    d0 = z - g0
    d1 = z - g1
    e0 = jnp.exp(a * d0 * d0)
    e1 = jnp.exp(a * d1 * d1)
    sil = z * jax.nn.sigmoid(z)
    o = jnp.dot(e0, wse_ref[...], precision="highest")
    o = o + jnp.dot(e1, wso_ref[...], precision="highest")
    o = o + jnp.dot(sil, wb_ref[...], precision="highest")
    return o


def _mean_norm(pp_ref, cc_ref):
    s = pp_ref[0] + pp_ref[1]
    c = jnp.maximum(cc_ref[0] + cc_ref[1], 1.0)
    h = s / c
    mu = jnp.mean(h, axis=1, keepdims=True)
    d = h - mu
    var = jnp.mean(d * d, axis=1, keepdims=True)
    return d * lax.rsqrt(var + 1e-5)


def _kan_body(params_ref, z_ref, wse_ref, wso_ref, wb_ref, out_ref):
    out_ref[...] = _kan_math(z_ref[...], params_ref, wse_ref, wso_ref, wb_ref)


def _norm_kan_body(params_ref, pp_ref, cc_ref, wse_ref, wso_ref, wb_ref, out_ref):
    z = _mean_norm(pp_ref, cc_ref)
    out_ref[...] = _kan_math(z, params_ref, wse_ref, wso_ref, wb_ref)


def _norm_kan_lsm_body(params_ref, pp_ref, cc_ref, wse_ref, wso_ref, wb_ref, out_ref):
    z = _mean_norm(pp_ref, cc_ref)
    o = _kan_math(z, params_ref, wse_ref, wso_ref, wb_ref)
    m = jnp.max(o, axis=1, keepdims=True)
    e = o - m
    out_ref[...] = e - jnp.log(jnp.sum(jnp.exp(e), axis=1, keepdims=True))


_W_SPEC = pl.BlockSpec((D, D), lambda i: (0, 0))
_ROW_SPEC = pl.BlockSpec((BN, D), lambda i: (i, 0))
_P_SPEC = pl.BlockSpec(memory_space=pltpu.SMEM)


def _kan_call(params, z, wse, wso, wb):
    return pl.pallas_call(
        _kan_body,
        grid=(GRID,),
        in_specs=[_P_SPEC, _ROW_SPEC, _W_SPEC, _W_SPEC, _W_SPEC],
        out_specs=_ROW_SPEC,
        out_shape=jax.ShapeDtypeStruct((N, D), jnp.float32),
    )(params, z, wse, wso, wb)


def _norm_kan_call(body, params, pp, cc, wse, wso, wb):
    return pl.pallas_call(
        body,
        grid=(GRID,),
        in_specs=[
            _P_SPEC,
            pl.BlockSpec((NC, BN, D), lambda i: (0, i, 0)),
            pl.BlockSpec((NC, BN, 1), lambda i: (0, i, 0)),
            _W_SPEC, _W_SPEC, _W_SPEC,
        ],
        out_specs=_ROW_SPEC,
        out_shape=jax.ShapeDtypeStruct((N, D), jnp.float32),
    )(params, pp, cc, wse, wso, wb)


@functools.lru_cache(maxsize=1)
def _make_seg():
    mesh = plsc.VectorSubcoreMesh(
        core_axis_name="c", subcore_axis_name="s", num_cores=NC, num_subcores=NS
    )

    @functools.partial(
        pl.kernel,
        out_type=(
            jax.ShapeDtypeStruct((NC, NPAD, D), jnp.float32),
            jax.ShapeDtypeStruct((NC * NPAD,), jnp.float32),
        ),
        mesh=mesh,
        scratch_types=[
            pltpu.VMEM((CH, 128), jnp.int32),
            pltpu.VMEM((CH, 128), jnp.int32),
            pltpu.VMEM((128, D), jnp.float32),
            pltpu.VMEM((128,), jnp.float32),
            pltpu.VMEM((RPT,), jnp.float32),
            pltpu.VMEM_SHARED((NPAD, D), jnp.float32),
            pltpu.VMEM_SHARED((NPAD,), jnp.float32),
            pltpu.SemaphoreType.DMA,
        ],
    )
    def seg(m_hbm, src_hbm, dst_hbm, za_hbm, zc_hbm, ones_hbm,
            agg_out, cnt_out,
            src_v, dst_v, rows_v, ones_v, cstg_v, agg_sp, cnt_sp, sem):
        cid = lax.axis_index("c")
        sid = lax.axis_index("s")
        w = cid * NS + sid
        r0 = sid * RPT
        pltpu.sync_copy(za_hbm.at[pl.ds(r0, RPT)], agg_sp.at[pl.ds(r0, RPT)])
        pltpu.sync_copy(zc_hbm.at[pl.ds(r0, RPT)], cstg_v)
        pltpu.sync_copy(cstg_v, cnt_sp.at[pl.ds(r0, RPT)])
        pltpu.sync_copy(ones_hbm, ones_v)
        pltpu.sync_copy(src_hbm.at[w], src_v)
        pltpu.sync_copy(dst_hbm.at[w], dst_v)
        plsc.subcore_barrier()

        def body(j, carry):
            pltpu.async_copy(m_hbm.at[src_v.at[j]], rows_v, sem).wait()
            pltpu.sync_copy(rows_v, agg_sp.at[dst_v.at[j]], add=True)
            pltpu.sync_copy(ones_v, cnt_sp.at[dst_v.at[j]], add=True)
            return carry

        lax.fori_loop(0, CH, body, 0)
        plsc.subcore_barrier()
        pltpu.sync_copy(agg_sp.at[pl.ds(r0, RPT)], agg_out.at[cid, pl.ds(r0, RPT)])
        pltpu.sync_copy(cnt_sp.at[pl.ds(r0, RPT)], cstg_v)
        pltpu.sync_copy(cstg_v, cnt_out.at[pl.ds(cid * NPAD + r0, RPT)])

    return seg


def _params_of(grid):
    g0 = grid[0]
    g1 = grid[-1]
    denom = (g1 - g0) / (grid.shape[0] - 1)
    return jnp.stack([g0, g1, -1.0 / (denom * denom)])


def kernel(x, edge_index, grid1, Ws1, Wb1, grid2, Ws2, Wb2, grid_out, Ws_out, Wb_out):
    src, dst = edge_index[0], edge_index[1]
    loops = jnp.arange(N, dtype=jnp.int32)
    pad = EPAD - (src.shape[0] + N)
    srcs = jnp.concatenate(
        [src, loops, jnp.zeros((pad,), jnp.int32)]).reshape(NW, CH, 128)
    dsts = jnp.concatenate(
        [dst, loops, jnp.full((pad,), DUMMY, jnp.int32)]).reshape(NW, CH, 128)
    za = jnp.zeros((NPAD, D), jnp.float32)
    zc = jnp.zeros((NPAD,), jnp.float32)
    ones = jnp.ones((128,), jnp.float32)
    seg = _make_seg()

    m1 = _kan_call(_params_of(grid1), x, Ws1[0::2], Ws1[1::2], Wb1)
    agg1, cnt1 = seg(m1, srcs, dsts, za, zc, ones)
    m2 = _norm_kan_call(_norm_kan_body, _params_of(grid2), agg1,
                        cnt1.reshape(NC, NPAD, 1), Ws2[0::2], Ws2[1::2], Wb2)
    agg2, cnt2 = seg(m2, srcs, dsts, za, zc, ones)
    return _norm_kan_call(_norm_kan_lsm_body, _params_of(grid_out), agg2,
                          cnt2.reshape(NC, NPAD, 1), Ws_out[0::2], Ws_out[1::2], Wb_out)


# speedup vs baseline: 2.6850x; 1.0794x over previous
import functools

import jax
import jax.numpy as jnp
from jax import lax
from jax.experimental import pallas as pl
from jax.experimental.pallas import tpu as pltpu
from jax.experimental.pallas import tpu_sc as plsc

N = 10000
D = 128
NC = 2
NS = 16
NW = NC * NS
CH = 81
EPW = CH * 128
EPAD = NW * EPW
RPT = 632
NPAD = NS * RPT
DUMMY = N
BN = 1000
GRID = N // BN


def _kan_math(z, params_ref, wse_ref, wso_ref, wb_ref):
    g0 = params_ref[0]
    g1 = params_ref[1]
    a = params_ref[2]
    d0 = z - g0
    d1 = z - g1
    e0 = jnp.exp(a * d0 * d0)
    e1 = jnp.exp(a * d1 * d1)
    sil = z * jax.nn.sigmoid(z)
    o = jnp.dot(e0, wse_ref[...])
    o = o + jnp.dot(e1, wso_ref[...])
    o = o + jnp.dot(sil, wb_ref[...])
    return o


def _mean_norm(pp_ref, cc_ref):
    s = pp_ref[0] + pp_ref[1]
    c = jnp.maximum(cc_ref[0] + cc_ref[1], 1.0)
    h = s / c
    mu = jnp.mean(h, axis=1, keepdims=True)
    d = h - mu
    var = jnp.mean(d * d, axis=1, keepdims=True)
    return d * lax.rsqrt(var + 1e-5)


def _kan_body(params_ref, z_ref, wse_ref, wso_ref, wb_ref, out_ref):
    out_ref[...] = _kan_math(z_ref[...], params_ref, wse_ref, wso_ref, wb_ref)


def _norm_kan_body(params_ref, pp_ref, cc_ref, wse_ref, wso_ref, wb_ref, out_ref):
    z = _mean_norm(pp_ref, cc_ref)
    out_ref[...] = _kan_math(z, params_ref, wse_ref, wso_ref, wb_ref)


def _norm_kan_lsm_body(params_ref, pp_ref, cc_ref, wse_ref, wso_ref, wb_ref, out_ref):
    z = _mean_norm(pp_ref, cc_ref)
    o = _kan_math(z, params_ref, wse_ref, wso_ref, wb_ref)
    m = jnp.max(o, axis=1, keepdims=True)
    e = o - m
    out_ref[...] = e - jnp.log(jnp.sum(jnp.exp(e), axis=1, keepdims=True))


_W_SPEC = pl.BlockSpec((D, D), lambda i: (0, 0))
_ROW_SPEC = pl.BlockSpec((BN, D), lambda i: (i, 0))
_P_SPEC = pl.BlockSpec(memory_space=pltpu.SMEM)


def _kan_call(params, z, wse, wso, wb):
    return pl.pallas_call(
        _kan_body,
        grid=(GRID,),
        in_specs=[_P_SPEC, _ROW_SPEC, _W_SPEC, _W_SPEC, _W_SPEC],
        out_specs=_ROW_SPEC,
        out_shape=jax.ShapeDtypeStruct((N, D), jnp.float32),
    )(params, z, wse, wso, wb)


def _norm_kan_call(body, params, pp, cc, wse, wso, wb):
    return pl.pallas_call(
        body,
        grid=(GRID,),
        in_specs=[
            _P_SPEC,
            pl.BlockSpec((NC, BN, D), lambda i: (0, i, 0)),
            pl.BlockSpec((NC, BN, 1), lambda i: (0, i, 0)),
            _W_SPEC, _W_SPEC, _W_SPEC,
        ],
        out_specs=_ROW_SPEC,
        out_shape=jax.ShapeDtypeStruct((N, D), jnp.float32),
    )(params, pp, cc, wse, wso, wb)


@functools.lru_cache(maxsize=1)
def _make_seg():
    mesh = plsc.VectorSubcoreMesh(
        core_axis_name="c", subcore_axis_name="s", num_cores=NC, num_subcores=NS
    )

    @functools.partial(
        pl.kernel,
        out_type=(
            jax.ShapeDtypeStruct((NC, NPAD, D), jnp.float32),
            jax.ShapeDtypeStruct((NC * NPAD,), jnp.float32),
        ),
        mesh=mesh,
        scratch_types=[
            pltpu.VMEM((CH, 128), jnp.int32),
            pltpu.VMEM((CH, 128), jnp.int32),
            pltpu.VMEM((128, D), jnp.float32),
            pltpu.VMEM((128,), jnp.float32),
            pltpu.VMEM((RPT,), jnp.float32),
            pltpu.VMEM_SHARED((NPAD, D), jnp.float32),
            pltpu.VMEM_SHARED((NPAD,), jnp.float32),
            pltpu.SemaphoreType.DMA,
        ],
    )
    def seg(m_hbm, src_hbm, dst_hbm, za_hbm, zc_hbm, ones_hbm,
            agg_out, cnt_out,
            src_v, dst_v, rows_v, ones_v, cstg_v, agg_sp, cnt_sp, sem):
        cid = lax.axis_index("c")
        sid = lax.axis_index("s")
        w = cid * NS + sid
        r0 = sid * RPT
        pltpu.sync_copy(za_hbm.at[pl.ds(r0, RPT)], agg_sp.at[pl.ds(r0, RPT)])
        pltpu.sync_copy(zc_hbm.at[pl.ds(r0, RPT)], cstg_v)
        pltpu.sync_copy(cstg_v, cnt_sp.at[pl.ds(r0, RPT)])
        pltpu.sync_copy(ones_hbm, ones_v)
        pltpu.sync_copy(src_hbm.at[w], src_v)
        pltpu.sync_copy(dst_hbm.at[w], dst_v)
        plsc.subcore_barrier()

        def body(j, carry):
            pltpu.async_copy(m_hbm.at[src_v.at[j]], rows_v, sem).wait()
            pltpu.sync_copy(rows_v, agg_sp.at[dst_v.at[j]], add=True)
            pltpu.sync_copy(ones_v, cnt_sp.at[dst_v.at[j]], add=True)
            return carry

        lax.fori_loop(0, CH, body, 0)
        plsc.subcore_barrier()
        pltpu.sync_copy(agg_sp.at[pl.ds(r0, RPT)], agg_out.at[cid, pl.ds(r0, RPT)])
        pltpu.sync_copy(cnt_sp.at[pl.ds(r0, RPT)], cstg_v)
        pltpu.sync_copy(cstg_v, cnt_out.at[pl.ds(cid * NPAD + r0, RPT)])

    return seg


def _params_of(grid):
    g0 = grid[0]
    g1 = grid[-1]
    denom = (g1 - g0) / (grid.shape[0] - 1)
    return jnp.stack([g0, g1, -1.0 / (denom * denom)])


def kernel(x, edge_index, grid1, Ws1, Wb1, grid2, Ws2, Wb2, grid_out, Ws_out, Wb_out):
    src, dst = edge_index[0], edge_index[1]
    loops = jnp.arange(N, dtype=jnp.int32)
    pad = EPAD - (src.shape[0] + N)
    srcs = jnp.concatenate(
        [src, loops, jnp.zeros((pad,), jnp.int32)]).reshape(NW, CH, 128)
    dsts = jnp.concatenate(
        [dst, loops, jnp.full((pad,), DUMMY, jnp.int32)]).reshape(NW, CH, 128)
    za = jnp.zeros((NPAD, D), jnp.float32)
    zc = jnp.zeros((NPAD,), jnp.float32)
    ones = jnp.ones((128,), jnp.float32)
    seg = _make_seg()

    m1 = _kan_call(_params_of(grid1), x, Ws1[0::2], Ws1[1::2], Wb1)
    agg1, cnt1 = seg(m1, srcs, dsts, za, zc, ones)
    m2 = _norm_kan_call(_norm_kan_body, _params_of(grid2), agg1,
                        cnt1.reshape(NC, NPAD, 1), Ws2[0::2], Ws2[1::2], Wb2)
    agg2, cnt2 = seg(m2, srcs, dsts, za, zc, ones)
    return _norm_kan_call(_norm_kan_lsm_body, _params_of(grid_out), agg2,
                          cnt2.reshape(NC, NPAD, 1), Ws_out[0::2], Ws_out[1::2], Wb_out)


# speedup vs baseline: 2.8251x; 1.0522x over previous
import functools

import jax
import jax.numpy as jnp
from jax import lax
from jax.experimental import pallas as pl
from jax.experimental.pallas import tpu as pltpu
from jax.experimental.pallas import tpu_sc as plsc

N = 10000
D = 128
NC = 2
NS = 16
NW = NC * NS
CH = 81
EPW = CH * 128
EPAD = NW * EPW
RPT = 632
NPAD = NS * RPT
DUMMY = N
BN = 1000
GRID = N // BN


def _kan_math(z, params_ref, wse_ref, wso_ref, wb_ref):
    g0 = params_ref[0]
    g1 = params_ref[1]
    a = params_ref[2]
    d0 = z - g0
    d1 = z - g1
    e0 = jnp.exp(a * d0 * d0)
    e1 = jnp.exp(a * d1 * d1)
    sil = z * jax.nn.sigmoid(z)
    o = jnp.dot(e0, wse_ref[...])
    o = o + jnp.dot(e1, wso_ref[...])
    o = o + jnp.dot(sil, wb_ref[...])
    return o


def _mean_norm(pp_ref, cc_ref):
    s = pp_ref[0] + pp_ref[1]
    c = jnp.maximum(cc_ref[0] + cc_ref[1], 1.0)
    h = s / c
    mu = jnp.mean(h, axis=1, keepdims=True)
    d = h - mu
    var = jnp.mean(d * d, axis=1, keepdims=True)
    return d * lax.rsqrt(var + 1e-5)


def _kan_body(params_ref, z_ref, wse_ref, wso_ref, wb_ref, out_ref):
    out_ref[...] = _kan_math(z_ref[...], params_ref, wse_ref, wso_ref, wb_ref)


def _norm_kan_body(params_ref, pp_ref, cc_ref, wse_ref, wso_ref, wb_ref, out_ref):
    z = _mean_norm(pp_ref, cc_ref)
    out_ref[...] = _kan_math(z, params_ref, wse_ref, wso_ref, wb_ref)


def _norm_kan_lsm_body(params_ref, pp_ref, cc_ref, wse_ref, wso_ref, wb_ref, out_ref):
    z = _mean_norm(pp_ref, cc_ref)
    o = _kan_math(z, params_ref, wse_ref, wso_ref, wb_ref)
    m = jnp.max(o, axis=1, keepdims=True)
    e = o - m
    out_ref[...] = e - jnp.log(jnp.sum(jnp.exp(e), axis=1, keepdims=True))


_W_SPEC = pl.BlockSpec((D, D), lambda i: (0, 0))
_ROW_SPEC = pl.BlockSpec((BN, D), lambda i: (i, 0))
_P_SPEC = pl.BlockSpec(memory_space=pltpu.SMEM)


def _kan_call(params, z, wse, wso, wb):
    return pl.pallas_call(
        _kan_body,
        grid=(GRID,),
        in_specs=[_P_SPEC, _ROW_SPEC, _W_SPEC, _W_SPEC, _W_SPEC],
        out_specs=_ROW_SPEC,
        out_shape=jax.ShapeDtypeStruct((N, D), jnp.float32),
    )(params, z, wse, wso, wb)


def _norm_kan_call(body, params, pp, cc, wse, wso, wb):
    return pl.pallas_call(
        body,
        grid=(GRID,),
        in_specs=[
            _P_SPEC,
            pl.BlockSpec((NC, BN, D), lambda i: (0, i, 0)),
            pl.BlockSpec((NC, BN, 1), lambda i: (0, i, 0)),
            _W_SPEC, _W_SPEC, _W_SPEC,
        ],
        out_specs=_ROW_SPEC,
        out_shape=jax.ShapeDtypeStruct((N, D), jnp.float32),
    )(params, pp, cc, wse, wso, wb)


@functools.lru_cache(maxsize=2)
def _make_seg(with_cnt):
    mesh = plsc.VectorSubcoreMesh(
        core_axis_name="c", subcore_axis_name="s", num_cores=NC, num_subcores=NS
    )

    out_type = [jax.ShapeDtypeStruct((NC, NPAD, D), jnp.float32)]
    scratch = [
        pltpu.VMEM((CH, 128), jnp.int32),
        pltpu.VMEM((CH, 128), jnp.int32),
        pltpu.VMEM((128, D), jnp.float32),
        pltpu.VMEM_SHARED((NPAD, D), jnp.float32),
        pltpu.SemaphoreType.DMA,
    ]
    if with_cnt:
        out_type.append(jax.ShapeDtypeStruct((NC * NPAD,), jnp.float32))
        scratch += [
            pltpu.VMEM((128,), jnp.float32),
            pltpu.VMEM((RPT,), jnp.float32),
            pltpu.VMEM_SHARED((NPAD,), jnp.float32),
        ]

    @functools.partial(
        pl.kernel, out_type=tuple(out_type), mesh=mesh,
        scratch_types=scratch,
    )
    def seg(m_hbm, src_hbm, dst_hbm, za_hbm, zc_hbm, ones_hbm,
            agg_out, *rest):
        if with_cnt:
            cnt_out, src_v, dst_v, rows_v, agg_sp, sem, ones_v, cstg_v, cnt_sp = rest
        else:
            src_v, dst_v, rows_v, agg_sp, sem = rest
        cid = lax.axis_index("c")
        sid = lax.axis_index("s")
        w = cid * NS + sid
        r0 = sid * RPT
        pltpu.sync_copy(za_hbm.at[pl.ds(r0, RPT)], agg_sp.at[pl.ds(r0, RPT)])
        if with_cnt:
            pltpu.sync_copy(zc_hbm.at[pl.ds(r0, RPT)], cstg_v)
            pltpu.sync_copy(cstg_v, cnt_sp.at[pl.ds(r0, RPT)])
            pltpu.sync_copy(ones_hbm, ones_v)
        pltpu.sync_copy(src_hbm.at[w], src_v)
        pltpu.sync_copy(dst_hbm.at[w], dst_v)
        plsc.subcore_barrier()

        def body(j, carry):
            pltpu.async_copy(m_hbm.at[src_v.at[j]], rows_v, sem).wait()
            pltpu.sync_copy(rows_v, agg_sp.at[dst_v.at[j]], add=True)
            if with_cnt:
                pltpu.sync_copy(ones_v, cnt_sp.at[dst_v.at[j]], add=True)
            return carry

        lax.fori_loop(0, CH, body, 0)
        plsc.subcore_barrier()
        pltpu.sync_copy(agg_sp.at[pl.ds(r0, RPT)], agg_out.at[cid, pl.ds(r0, RPT)])
        if with_cnt:
            pltpu.sync_copy(cnt_sp.at[pl.ds(r0, RPT)], cstg_v)
            pltpu.sync_copy(cstg_v, cnt_out.at[pl.ds(cid * NPAD + r0, RPT)])

    return seg


def _params_of(grid):
    g0 = grid[0]
    g1 = grid[-1]
    denom = (g1 - g0) / (grid.shape[0] - 1)
    return jnp.stack([g0, g1, -1.0 / (denom * denom)])


def kernel(x, edge_index, grid1, Ws1, Wb1, grid2, Ws2, Wb2, grid_out, Ws_out, Wb_out):
    src, dst = edge_index[0], edge_index[1]
    loops = jnp.arange(N, dtype=jnp.int32)
    pad = EPAD - (src.shape[0] + N)
    srcs = jnp.concatenate(
        [src, loops, jnp.zeros((pad,), jnp.int32)]).reshape(NW, CH, 128)
    dsts = jnp.concatenate(
        [dst, loops, jnp.full((pad,), DUMMY, jnp.int32)]).reshape(NW, CH, 128)
    za = jnp.zeros((NPAD, D), jnp.float32)
    zc = jnp.zeros((NPAD,), jnp.float32)
    ones = jnp.ones((128,), jnp.float32)

    m1 = _kan_call(_params_of(grid1), x, Ws1[0::2], Ws1[1::2], Wb1)
    agg1, cnt1 = _make_seg(True)(m1, srcs, dsts, za, zc, ones)
    cnt = cnt1.reshape(NC, NPAD, 1)
    m2 = _norm_kan_call(_norm_kan_body, _params_of(grid2), agg1,
                        cnt, Ws2[0::2], Ws2[1::2], Wb2)
    (agg2,) = _make_seg(False)(m2, srcs, dsts, za, zc, ones)
    return _norm_kan_call(_norm_kan_lsm_body, _params_of(grid_out), agg2,
                          cnt, Ws_out[0::2], Ws_out[1::2], Wb_out)


# speedup vs baseline: 2.9045x; 1.0281x over previous
import functools

import jax
import jax.numpy as jnp
from jax import lax
from jax.experimental import pallas as pl
from jax.experimental.pallas import tpu as pltpu
from jax.experimental.pallas import tpu_sc as plsc

N = 10000
D = 128
NC = 2
NS = 16
NW = NC * NS
CH = 81
EPW = CH * 128
EPAD = NW * EPW
RPT = 632
NPAD = NS * RPT
DUMMY = N
BN = 2000
GRID = N // BN


def _kan_math(z, params_ref, wse_ref, wb_ref):
    g0 = params_ref[0]
    g1 = params_ref[1]
    a = params_ref[2]
    d0 = z - g0
    d1 = z - g1
    e01 = jnp.concatenate([jnp.exp(a * d0 * d0), jnp.exp(a * d1 * d1)], axis=1)
    sil = z * jax.nn.sigmoid(z)
    o = jnp.dot(e01, wse_ref[...])
    o = o + jnp.dot(sil, wb_ref[...])
    return o


def _mean_norm(pp_ref, cc_ref):
    s = pp_ref[0] + pp_ref[1]
    c = jnp.maximum(cc_ref[0] + cc_ref[1], 1.0)
    h = s / c
    mu = jnp.mean(h, axis=1, keepdims=True)
    d = h - mu
    var = jnp.mean(d * d, axis=1, keepdims=True)
    return d * lax.rsqrt(var + 1e-5)


def _kan_body(params_ref, z_ref, wse_ref, wb_ref, out_ref):
    out_ref[...] = _kan_math(z_ref[...], params_ref, wse_ref, wb_ref)


def _norm_kan_body(params_ref, pp_ref, cc_ref, wse_ref, wb_ref, out_ref):
    z = _mean_norm(pp_ref, cc_ref)
    out_ref[...] = _kan_math(z, params_ref, wse_ref, wb_ref)


def _norm_kan_lsm_body(params_ref, pp_ref, cc_ref, wse_ref, wb_ref, out_ref):
    z = _mean_norm(pp_ref, cc_ref)
    o = _kan_math(z, params_ref, wse_ref, wb_ref)
    m = jnp.max(o, axis=1, keepdims=True)
    e = o - m
    out_ref[...] = e - jnp.log(jnp.sum(jnp.exp(e), axis=1, keepdims=True))


_W_SPEC = pl.BlockSpec((D, D), lambda i: (0, 0))
_W2_SPEC = pl.BlockSpec((2 * D, D), lambda i: (0, 0))
_ROW_SPEC = pl.BlockSpec((BN, D), lambda i: (i, 0))
_P_SPEC = pl.BlockSpec(memory_space=pltpu.SMEM)


def _kan_call(params, z, wse, wb):
    return pl.pallas_call(
        _kan_body,
        grid=(GRID,),
        in_specs=[_P_SPEC, _ROW_SPEC, _W2_SPEC, _W_SPEC],
        out_specs=_ROW_SPEC,
        out_shape=jax.ShapeDtypeStruct((N, D), jnp.float32),
    )(params, z, wse, wb)


def _norm_kan_call(body, params, pp, cc, wse, wb):
    return pl.pallas_call(
        body,
        grid=(GRID,),
        in_specs=[
            _P_SPEC,
            pl.BlockSpec((NC, BN, D), lambda i: (0, i, 0)),
            pl.BlockSpec((NC, BN, 1), lambda i: (0, i, 0)),
            _W2_SPEC, _W_SPEC,
        ],
        out_specs=_ROW_SPEC,
        out_shape=jax.ShapeDtypeStruct((N, D), jnp.float32),
    )(params, pp, cc, wse, wb)


@functools.lru_cache(maxsize=2)
def _make_seg(with_cnt):
    mesh = plsc.VectorSubcoreMesh(
        core_axis_name="c", subcore_axis_name="s", num_cores=NC, num_subcores=NS
    )

    out_type = [jax.ShapeDtypeStruct((NC, NPAD, D), jnp.float32)]
    scratch = [
        pltpu.VMEM((CH, 128), jnp.int32),
        pltpu.VMEM((CH, 128), jnp.int32),
        pltpu.VMEM((128, D), jnp.float32),
        pltpu.VMEM_SHARED((NPAD, D), jnp.float32),
        pltpu.SemaphoreType.DMA,
    ]
    if with_cnt:
        out_type.append(jax.ShapeDtypeStruct((NC * NPAD,), jnp.float32))
        scratch += [
            pltpu.VMEM((128,), jnp.float32),
            pltpu.VMEM((RPT,), jnp.float32),
            pltpu.VMEM_SHARED((NPAD,), jnp.float32),
        ]

    @functools.partial(
        pl.kernel, out_type=tuple(out_type), mesh=mesh,
        scratch_types=scratch,
    )
    def seg(m_hbm, src_hbm, dst_hbm, za_hbm, zc_hbm, ones_hbm,
            agg_out, *rest):
        if with_cnt:
            cnt_out, src_v, dst_v, rows_v, agg_sp, sem, ones_v, cstg_v, cnt_sp = rest
        else:
            src_v, dst_v, rows_v, agg_sp, sem = rest
        cid = lax.axis_index("c")
        sid = lax.axis_index("s")
        w = cid * NS + sid
        r0 = sid * RPT
        pltpu.sync_copy(za_hbm.at[pl.ds(r0, RPT)], agg_sp.at[pl.ds(r0, RPT)])
        if with_cnt:
            pltpu.sync_copy(zc_hbm.at[pl.ds(r0, RPT)], cstg_v)
            pltpu.sync_copy(cstg_v, cnt_sp.at[pl.ds(r0, RPT)])
            pltpu.sync_copy(ones_hbm, ones_v)
        pltpu.sync_copy(src_hbm.at[w], src_v)
        pltpu.sync_copy(dst_hbm.at[w], dst_v)
        plsc.subcore_barrier()

        def body(j, carry):
            pltpu.async_copy(m_hbm.at[src_v.at[j]], rows_v, sem).wait()
            pltpu.sync_copy(rows_v, agg_sp.at[dst_v.at[j]], add=True)
            if with_cnt:
                pltpu.sync_copy(ones_v, cnt_sp.at[dst_v.at[j]], add=True)
            return carry

        lax.fori_loop(0, CH, body, 0)
        plsc.subcore_barrier()
        pltpu.sync_copy(agg_sp.at[pl.ds(r0, RPT)], agg_out.at[cid, pl.ds(r0, RPT)])
        if with_cnt:
            pltpu.sync_copy(cnt_sp.at[pl.ds(r0, RPT)], cstg_v)
            pltpu.sync_copy(cstg_v, cnt_out.at[pl.ds(cid * NPAD + r0, RPT)])

    return seg


def _params_of(grid):
    g0 = grid[0]
    g1 = grid[-1]
    denom = (g1 - g0) / (grid.shape[0] - 1)
    return jnp.stack([g0, g1, -1.0 / (denom * denom)])


def kernel(x, edge_index, grid1, Ws1, Wb1, grid2, Ws2, Wb2, grid_out, Ws_out, Wb_out):
    src, dst = edge_index[0], edge_index[1]
    loops = jnp.arange(N, dtype=jnp.int32)
    pad = EPAD - (src.shape[0] + N)
    srcs = jnp.concatenate(
        [src, loops, jnp.zeros((pad,), jnp.int32)]).reshape(NW, CH, 128)
    dsts = jnp.concatenate(
        [dst, loops, jnp.full((pad,), DUMMY, jnp.int32)]).reshape(NW, CH, 128)
    za = jnp.zeros((NPAD, D), jnp.float32)
    zc = jnp.zeros((NPAD,), jnp.float32)
    ones = jnp.ones((128,), jnp.float32)

    m1 = _kan_call(_params_of(grid1), x,
                   jnp.concatenate([Ws1[0::2], Ws1[1::2]], axis=0), Wb1)
    agg1, cnt1 = _make_seg(True)(m1, srcs, dsts, za, zc, ones)
    cnt = cnt1.reshape(NC, NPAD, 1)
    m2 = _norm_kan_call(_norm_kan_body, _params_of(grid2), agg1, cnt,
                        jnp.concatenate([Ws2[0::2], Ws2[1::2]], axis=0), Wb2)
    (agg2,) = _make_seg(False)(m2, srcs, dsts, za, zc, ones)
    return _norm_kan_call(_norm_kan_lsm_body, _params_of(grid_out), agg2, cnt,
                          jnp.concatenate([Ws_out[0::2], Ws_out[1::2]], axis=0),
                          Wb_out)
